# baseline (speedup 1.0000x reference)
import functools

import jax
import jax.numpy as jnp
from jax import lax
from jax.experimental import pallas as pl
from jax.experimental.pallas import tpu as pltpu
from jax.experimental.pallas import tpu_sc as plsc

NC, NS = 2, 16
NW = NC * NS
CH = 32
CH2 = CH // 2
L = 16


@functools.partial(jax.jit, static_argnames=("n_chunks", "dim"))
def _sc_gather(idx, table4, n_chunks, dim):
    d4 = 4 * dim
    d2 = 2 * dim
    nv = dim // L
    tail_src = dim - L

    @functools.partial(
        pl.kernel,
        out_type=jax.ShapeDtypeStruct((NW * n_chunks * CH2, d2), jnp.float32),
        mesh=plsc.VectorSubcoreMesh(core_axis_name="c", subcore_axis_name="s"),
        compiler_params=pltpu.CompilerParams(use_tc_tiling_on_sc=False),
        scratch_types=[
            pltpu.VMEM((n_chunks, CH), jnp.int32),
            pltpu.VMEM((n_chunks, CH), jnp.int32),
            pltpu.VMEM((n_chunks, CH), jnp.int32),
            pltpu.VMEM((CH, d4), jnp.float32),
            pltpu.VMEM((CH, d4), jnp.float32),
            pltpu.VMEM((CH2, d2), jnp.float32),
            pltpu.SemaphoreType.DMA,
            pltpu.SemaphoreType.DMA,
        ],
    )
    def k(idx_hbm, t4_hbm, out_hbm, idx_v, g_v, off_v, raw0, raw1, comp, s0, s1):
        wid = lax.axis_index("s") * NC + lax.axis_index("c")
        pltpu.sync_copy(idx_hbm.at[wid], idx_v)

        def pre(r, carry):
            for h in range(CH // L):
                v = idx_v.at[r][pl.ds(h * L, L)]
                g_v.at[r][pl.ds(h * L, L)] = jax.lax.shift_right_logical(v, 2)
                off_v.at[r][pl.ds(h * L, L)] = (v & 3) * dim
            return carry

        lax.fori_loop(0, n_chunks, pre, 0)

        def gather(j, buf, sem):
            return pltpu.make_async_copy(t4_hbm.at[g_v.at[j]], buf, sem)

        def process(j, buf):
            offs = [off_v.at[j][pl.ds(h * L, L)] for h in range(CH // L)]
            for l in range(CH):
                off = offs[l // L][l % L]
                p, c0 = l // 2, dim * (l % 2)
                for t in range(nv):
                    comp[p, pl.ds(c0 + t * L, L)] = buf[l, pl.ds(off + t * L, L)]
                comp[p, pl.ds(c0 + tail_src, L)] = buf[l, pl.ds(off + tail_src, L)]
            pltpu.sync_copy(
                comp, out_hbm.at[pl.ds((wid * n_chunks + j) * CH2, CH2)]
            )

        gather(0, raw0, s0).start()

        def body(t, carry):
            j = 2 * t
            gather(j + 1, raw1, s1).start()
            gather(j, raw0, s0).wait()
            process(j, raw0)

            @pl.when(t < n_chunks // 2 - 1)
            def _():
                gather(j + 2, raw0, s0).start()

            gather(j + 1, raw1, s1).wait()
            process(j + 1, raw1)
            return carry

        lax.fori_loop(0, n_chunks // 2, body, 0)

    return k(idx, table4)


def kernel(indices, table):
    batch, seq = indices.shape
    vocab, dim = table.shape
    total = batch * seq
    assert total % (NW * CH) == 0 and vocab % 4 == 0 and dim % 4 == 0
    n_chunks = total // (NW * CH)
    idx = indices.reshape(NW, n_chunks, CH)
    table4 = table.reshape(vocab // 4, 4 * dim)
    out = _sc_gather(idx, table4, n_chunks, dim)
    return out.reshape(batch, seq, dim)


# speedup vs baseline: 1.6250x; 1.6250x over previous
import functools

import jax
import jax.numpy as jnp
from jax import lax
from jax.experimental import pallas as pl
from jax.experimental.pallas import tpu as pltpu
from jax.experimental.pallas import tpu_sc as plsc

NC, NS = 2, 16
NW = NC * NS
CHUNK = 128
DP = 304
RB = 40
L = 16


def _repack(table, vocab, dim):
    n_blocks = vocab // RB
    blk = RB * DP
    nfull = dim // L
    tail = dim - L

    @functools.partial(
        pl.kernel,
        out_type=jax.ShapeDtypeStruct((vocab * DP,), jnp.float32),
        mesh=plsc.VectorSubcoreMesh(core_axis_name="c", subcore_axis_name="s"),
        compiler_params=pltpu.CompilerParams(use_tc_tiling_on_sc=True),
        scratch_types=[
            pltpu.VMEM((RB, dim), jnp.float32),
            pltpu.VMEM((RB, dim), jnp.float32),
            pltpu.VMEM((blk,), jnp.float32),
            pltpu.SemaphoreType.DMA,
            pltpu.SemaphoreType.DMA,
        ],
    )
    def k(tbl_hbm, out_hbm, in0, in1, lin, s0, s1):
        wid = lax.axis_index("s") * NC + lax.axis_index("c")

        def read(b, buf, sem):
            return pltpu.make_async_copy(tbl_hbm.at[pl.ds(RB * b, RB)], buf, sem)

        def proc(b, buf):
            for l in range(RB):
                base = DP * l
                for t in range(nfull):
                    lin[pl.ds(base + t * L, L)] = buf[l, pl.ds(t * L, L)]
                lin[pl.ds(base + tail, L)] = buf[l, pl.ds(tail, L)]
            pltpu.sync_copy(lin, out_hbm.at[pl.ds(blk * b, blk)])

        read(wid, in0, s0).start()

        def body(t, carry):
            b0 = wid + NW * 2 * t
            b1, b2 = b0 + NW, b0 + 2 * NW

            @pl.when(b1 < n_blocks)
            def _():
                read(b1, in1, s1).start()

            read(b0, in0, s0).wait()
            proc(b0, in0)

            @pl.when(b2 < n_blocks)
            def _():
                read(b2, in0, s0).start()

            @pl.when(b1 < n_blocks)
            def _():
                read(b1, in1, s1).wait()
                proc(b1, in1)

            return carry

        n_pairs = (n_blocks - wid + 2 * NW - 1) // (2 * NW)
        lax.fori_loop(0, n_pairs, body, 0)

    return k(table)


def _gather(idx, table_pad, n_chunks):
    @functools.partial(
        pl.kernel,
        out_type=jax.ShapeDtypeStruct((NW * n_chunks * CHUNK, DP), jnp.float32),
        mesh=plsc.VectorSubcoreMesh(core_axis_name="c", subcore_axis_name="s"),
        compiler_params=pltpu.CompilerParams(use_tc_tiling_on_sc=False),
        scratch_types=[
            pltpu.VMEM((n_chunks, CHUNK), jnp.int32),
            pltpu.VMEM((CHUNK, DP), jnp.float32),
            pltpu.VMEM((CHUNK, DP), jnp.float32),
            pltpu.SemaphoreType.DMA,
            pltpu.SemaphoreType.DMA,
        ],
    )
    def k(idx_hbm, tbl_hbm, out_hbm, idx_v, buf0, buf1, g0, g1):
        wid = lax.axis_index("s") * NC + lax.axis_index("c")
        base = wid * n_chunks * CHUNK
        pltpu.sync_copy(idx_hbm.at[wid], idx_v)

        def gather(j, buf, sem):
            return pltpu.make_async_copy(tbl_hbm.at[idx_v.at[j]], buf, sem)

        def scatter(j, buf):
            pltpu.sync_copy(buf, out_hbm.at[pl.ds(base + j * CHUNK, CHUNK)])

        gather(0, buf0, g0).start()

        def body(t, carry):
            j = 2 * t
            gather(j + 1, buf1, g1).start()
            gather(j, buf0, g0).wait()
            scatter(j, buf0)

            @pl.when(t < n_chunks // 2 - 1)
            def _():
                gather(j + 2, buf0, g0).start()

            gather(j + 1, buf1, g1).wait()
            scatter(j + 1, buf1)
            return carry

        lax.fori_loop(0, n_chunks // 2, body, 0)

    return k(idx, table_pad)


def kernel(indices, table):
    batch, seq = indices.shape
    vocab, dim = table.shape
    total = batch * seq
    assert total % (NW * CHUNK) == 0 and vocab % RB == 0 and dim <= DP
    n_chunks = total // (NW * CHUNK)
    idx = indices.reshape(NW, n_chunks, CHUNK)
    lin = _repack(table, vocab, dim)
    table_pad = lin.reshape(vocab, DP)
    out = _gather(idx, table_pad, n_chunks)
    return out[:, :dim].reshape(batch, seq, dim)


# speedup vs baseline: 2.4359x; 1.4990x over previous
import functools

import jax
import jax.numpy as jnp
from jax import lax
from jax.experimental import pallas as pl
from jax.experimental.pallas import tpu as pltpu
from jax.experimental.pallas import tpu_sc as plsc

NC, NS = 2, 16
NW = NC * NS
RB = 40
L = 16
DP = 384


def _repack(table, vocab, dim):
    n_blocks = vocab // RB
    orows = RB * DP // 128
    nfull = dim // L
    tail = dim - L

    @functools.partial(
        pl.kernel,
        out_type=jax.ShapeDtypeStruct((vocab * DP // 128, 128), jnp.float32),
        mesh=plsc.VectorSubcoreMesh(core_axis_name="c", subcore_axis_name="s"),
        compiler_params=pltpu.CompilerParams(use_tc_tiling_on_sc=True),
        scratch_types=[
            pltpu.VMEM((RB, dim), jnp.float32),
            pltpu.VMEM((RB, dim), jnp.float32),
            pltpu.VMEM((orows, 128), jnp.float32),
            pltpu.SemaphoreType.DMA,
            pltpu.SemaphoreType.DMA,
        ],
    )
    def k(tbl_hbm, out_hbm, in0, in1, lin, s0, s1):
        wid = lax.axis_index("s") * NC + lax.axis_index("c")

        def read(b, buf, sem):
            return pltpu.make_async_copy(tbl_hbm.at[pl.ds(RB * b, RB)], buf, sem)

        def proc(b, buf):
            for l in range(RB):
                for t in range(nfull):
                    f = DP * l + t * L
                    lin[f // 128, pl.ds(f % 128, L)] = buf[l, pl.ds(t * L, L)]
                f = DP * l + tail
                lin[f // 128, pl.ds(f % 128, L)] = buf[l, pl.ds(tail, L)]
            pltpu.sync_copy(lin, out_hbm.at[pl.ds(orows * b, orows)])

        read(wid, in0, s0).start()

        def body(t, carry):
            b0 = wid + NW * 2 * t
            b1, b2 = b0 + NW, b0 + 2 * NW

            @pl.when(b1 < n_blocks)
            def _():
                read(b1, in1, s1).start()

            read(b0, in0, s0).wait()
            proc(b0, in0)

            @pl.when(b2 < n_blocks)
            def _():
                read(b2, in0, s0).start()

            @pl.when(b1 < n_blocks)
            def _():
                read(b1, in1, s1).wait()
                proc(b1, in1)

            return carry

        n_pairs = (n_blocks - wid + 2 * NW - 1) // (2 * NW)
        lax.fori_loop(0, n_pairs, body, 0)

    return k(table)


def _gather_emit(idxp, tblr, batch, seq, dim):
    n_chunks = batch // NW

    @functools.partial(
        pl.kernel,
        out_type=jax.ShapeDtypeStruct((batch, seq, dim), jnp.float32),
        mesh=plsc.VectorSubcoreMesh(core_axis_name="c", subcore_axis_name="s"),
        compiler_params=pltpu.CompilerParams(use_tc_tiling_on_sc=True),
        scratch_types=[
            pltpu.VMEM((n_chunks, seq), jnp.int32),
            pltpu.VMEM((3, seq), jnp.int32),
            pltpu.VMEM((3, seq), jnp.int32),
            pltpu.VMEM((seq, 128), jnp.float32),
            pltpu.VMEM((seq, 128), jnp.float32),
            pltpu.VMEM((seq, dim), jnp.float32),
            pltpu.VMEM((seq, dim), jnp.float32),
            [pltpu.SemaphoreType.DMA] * 3,
            [pltpu.SemaphoreType.DMA] * 3,
            pltpu.SemaphoreType.DMA,
            pltpu.SemaphoreType.DMA,
        ],
    )
    def k(idx_hbm, tbl_hbm, out_hbm, idx_v, i3a, i3b, s2a, s2b, im0, im1,
          ga, gb_, o0, o1):
        wid = lax.axis_index("s") * NC + lax.axis_index("c")
        pltpu.sync_copy(idx_hbm.at[wid], idx_v)

        def expand(j, i3):
            for lo in (0, seq - L):
                v = idx_v.at[j][pl.ds(lo, L)]
                b3 = 3 * v
                for ct in range(3):
                    i3[ct, pl.ds(lo, L)] = b3 + ct

        def tiles(img, s2):
            return [
                img.at[:, pl.ds(0, 128)],
                img.at[:, pl.ds(128, 128)],
                s2,
            ]

        def gathers(i3, img, s2, sem):
            for ct, dst in enumerate(tiles(img, s2)):
                pltpu.make_async_copy(
                    tbl_hbm.at[i3.at[ct]], dst, sem[ct]
                ).start()

        def gwait(i3, img, s2, sem):
            for ct, dst in enumerate(tiles(img, s2)):
                pltpu.make_async_copy(
                    tbl_hbm.at[i3.at[ct]], dst, sem[ct]
                ).wait()

        def assemble(j, s2, img, osem):
            for l in range(seq):
                img[l, pl.ds(256, L)] = s2[l, pl.ds(0, L)]
                img[l, pl.ds(dim - L, L)] = s2[l, pl.ds(dim - L - 256, L)]
                img[l, pl.ds(272, L)] = s2[l, pl.ds(16, L)]
            pltpu.make_async_copy(
                img, out_hbm.at[wid * n_chunks + j], osem
            ).start()

        def owait(img, osem):
            pltpu.make_async_copy(img, out_hbm.at[0], osem).wait()

        expand(0, i3a)
        gathers(i3a, im0, s2a, ga)

        def body(t, carry):
            j = 2 * t

            @pl.when(t > 0)
            def _():
                owait(im1, o1)

            expand(j + 1, i3b)
            gathers(i3b, im1, s2b, gb_)

            gwait(i3a, im0, s2a, ga)
            assemble(j, s2a, im0, o0)

            @pl.when(t < n_chunks // 2 - 1)
            def _():
                owait(im0, o0)
                expand(j + 2, i3a)
                gathers(i3a, im0, s2a, ga)

            gwait(i3b, im1, s2b, gb_)
            assemble(j + 1, s2b, im1, o1)
            return carry

        lax.fori_loop(0, n_chunks // 2, body, 0)
        owait(im0, o0)
        owait(im1, o1)

    return k(idxp, tblr)


def kernel(indices, table):
    batch, seq = indices.shape
    vocab, dim = table.shape
    assert batch % NW == 0 and vocab % RB == 0 and 256 < dim <= 300
    idxp = indices.reshape(NW, batch // NW, seq)
    tblr = _repack(table, vocab, dim)
    return _gather_emit(idxp, tblr, batch, seq, dim)


# speedup vs baseline: 2.4360x; 1.0000x over previous
import functools

import jax
import jax.numpy as jnp
from jax import lax
from jax.experimental import pallas as pl
from jax.experimental.pallas import tpu as pltpu
from jax.experimental.pallas import tpu_sc as plsc

NC, NS = 2, 16
NW = NC * NS
RB = 40
L = 16
DP = 384


def _repack(table, vocab, dim):
    n_blocks = vocab // RB
    orows = RB * DP // 128
    nfull = dim // L
    tail = dim - L

    @functools.partial(
        pl.kernel,
        out_type=jax.ShapeDtypeStruct((vocab * DP // 128, 128), jnp.float32),
        mesh=plsc.VectorSubcoreMesh(core_axis_name="c", subcore_axis_name="s"),
        compiler_params=pltpu.CompilerParams(use_tc_tiling_on_sc=True),
        scratch_types=[
            pltpu.VMEM((RB, dim), jnp.float32),
            pltpu.VMEM((RB, dim), jnp.float32),
            pltpu.VMEM((orows, 128), jnp.float32),
            pltpu.SemaphoreType.DMA,
            pltpu.SemaphoreType.DMA,
        ],
    )
    def k(tbl_hbm, out_hbm, in0, in1, lin, s0, s1):
        wid = lax.axis_index("s") * NC + lax.axis_index("c")

        def read(b, buf, sem):
            return pltpu.make_async_copy(tbl_hbm.at[pl.ds(RB * b, RB)], buf, sem)

        def proc(b, buf):
            for l in range(RB):
                for t in range(nfull):
                    f = DP * l + t * L
                    lin[f // 128, pl.ds(f % 128, L)] = buf[l, pl.ds(t * L, L)]
                f = DP * l + tail
                lin[f // 128, pl.ds(f % 128, L)] = buf[l, pl.ds(tail, L)]
            pltpu.sync_copy(lin, out_hbm.at[pl.ds(orows * b, orows)])

        read(wid, in0, s0).start()

        def body(t, carry):
            b0 = wid + NW * 2 * t
            b1, b2 = b0 + NW, b0 + 2 * NW

            @pl.when(b1 < n_blocks)
            def _():
                read(b1, in1, s1).start()

            read(b0, in0, s0).wait()
            proc(b0, in0)

            @pl.when(b2 < n_blocks)
            def _():
                read(b2, in0, s0).start()

            @pl.when(b1 < n_blocks)
            def _():
                read(b1, in1, s1).wait()
                proc(b1, in1)

            return carry

        n_pairs = (n_blocks - wid + 2 * NW - 1) // (2 * NW)
        lax.fori_loop(0, n_pairs, body, 0)

    return k(table)


def _gather_emit(idxp, tblr, batch, seq, dim):
    n_chunks = batch // NW

    @functools.partial(
        pl.kernel,
        out_type=jax.ShapeDtypeStruct((batch, seq, dim), jnp.float32),
        mesh=plsc.VectorSubcoreMesh(core_axis_name="c", subcore_axis_name="s"),
        compiler_params=pltpu.CompilerParams(use_tc_tiling_on_sc=True),
        scratch_types=[
            pltpu.VMEM((n_chunks, seq), jnp.int32),
            pltpu.VMEM((3, seq), jnp.int32),
            pltpu.VMEM((3, seq), jnp.int32),
            pltpu.VMEM((seq, 128), jnp.float32),
            pltpu.VMEM((seq, 128), jnp.float32),
            pltpu.VMEM((seq, dim), jnp.float32),
            pltpu.VMEM((seq, dim), jnp.float32),
            [pltpu.SemaphoreType.DMA] * 3,
            [pltpu.SemaphoreType.DMA] * 3,
            pltpu.SemaphoreType.DMA,
            pltpu.SemaphoreType.DMA,
        ],
    )
    def k(idx_hbm, tbl_hbm, out_hbm, idx_v, i3a, i3b, s2a, s2b, im0, im1,
          ga, gb_, o0, o1):
        wid = lax.axis_index("s") * NC + lax.axis_index("c")
        pltpu.sync_copy(idx_hbm.at[pl.ds(wid * n_chunks, n_chunks)], idx_v)

        def expand(j, i3):
            for lo in (0, seq - L):
                v = idx_v.at[j][pl.ds(lo, L)]
                b3 = 3 * v
                for ct in range(3):
                    i3[ct, pl.ds(lo, L)] = b3 + ct

        def tiles(img, s2):
            return [
                img.at[:, pl.ds(0, 128)],
                img.at[:, pl.ds(128, 128)],
                s2,
            ]

        def gathers(i3, img, s2, sem):
            for ct, dst in enumerate(tiles(img, s2)):
                pltpu.make_async_copy(
                    tbl_hbm.at[i3.at[ct]], dst, sem[ct]
                ).start()

        def gwait(i3, img, s2, sem):
            for ct, dst in enumerate(tiles(img, s2)):
                pltpu.make_async_copy(
                    tbl_hbm.at[i3.at[ct]], dst, sem[ct]
                ).wait()

        def assemble(j, s2, img, osem):
            for l in range(seq):
                img[l, pl.ds(256, L)] = s2[l, pl.ds(0, L)]
                img[l, pl.ds(dim - L, L)] = s2[l, pl.ds(dim - L - 256, L)]
                img[l, pl.ds(272, L)] = s2[l, pl.ds(16, L)]
            pltpu.make_async_copy(
                img, out_hbm.at[wid * n_chunks + j], osem
            ).start()

        def owait(img, osem):
            pltpu.make_async_copy(img, out_hbm.at[0], osem).wait()

        expand(0, i3a)
        gathers(i3a, im0, s2a, ga)

        def body(t, carry):
            j = 2 * t

            @pl.when(t > 0)
            def _():
                owait(im1, o1)

            expand(j + 1, i3b)
            gathers(i3b, im1, s2b, gb_)

            gwait(i3a, im0, s2a, ga)
            assemble(j, s2a, im0, o0)

            @pl.when(t < n_chunks // 2 - 1)
            def _():
                owait(im0, o0)
                expand(j + 2, i3a)
                gathers(i3a, im0, s2a, ga)

            gwait(i3b, im1, s2b, gb_)
            assemble(j + 1, s2b, im1, o1)
            return carry

        lax.fori_loop(0, n_chunks // 2, body, 0)
        owait(im0, o0)
        owait(im1, o1)

    return k(idxp, tblr)


def kernel(indices, table):
    batch, seq = indices.shape
    vocab, dim = table.shape
    assert batch % NW == 0 and vocab % RB == 0 and 256 < dim <= 300
    tblr = _repack(table, vocab, dim)
    return _gather_emit(indices, tblr, batch, seq, dim)
